# Initial kernel scaffold; baseline (speedup 1.0000x reference)
import functools

import jax
import jax.numpy as jnp
from jax import lax
from jax.experimental import pallas as pl
from jax.experimental.pallas import tpu as pltpu
from jax.experimental.pallas import tpu_sc as plsc

VOCAB = 100000
SEQ_LEN = 200
BATCH = 1024
NTOK = SEQ_LEN * BATCH

NUM_WORKERS = 32
BINS_PER_TILE = 3200
VOCAB_PAD = NUM_WORKERS * BINS_PER_TILE

CHUNK = 8192
NCHUNK = NTOK // CHUNK
VREGS_PER_CHUNK = CHUNK // 16

BW = 2048
GRID = (VOCAB + BW - 1) // BW


def _sc_histogram(src_flat):
    mesh = plsc.VectorSubcoreMesh(core_axis_name="c", subcore_axis_name="s")

    @functools.partial(
        pl.kernel,
        mesh=mesh,
        out_type=jax.ShapeDtypeStruct((VOCAB_PAD,), jnp.float32),
        scratch_types=[
            pltpu.VMEM((CHUNK,), jnp.int32),
            pltpu.VMEM((CHUNK,), jnp.int32),
            pltpu.VMEM((BINS_PER_TILE,), jnp.float32),
            pltpu.SemaphoreType.DMA,
            pltpu.SemaphoreType.DMA,
        ],
    )
    def hist_kernel(src_hbm, out_hbm, buf0, buf1, hist, sem0, sem1):
        c = lax.axis_index("c")
        s = lax.axis_index("s")
        wid = s * 2 + c
        base = wid * BINS_PER_TILE

        zeros16 = jnp.zeros((16,), jnp.float32)

        def zero_body(i, carry):
            hist[pl.ds(i * 16, 16)] = zeros16
            return carry

        lax.fori_loop(0, BINS_PER_TILE // 16, zero_body, 0)

        ones16 = jnp.ones((16,), jnp.float32)
        bufs = (buf0, buf1)
        sems = (sem0, sem1)

        copies = [None, None]
        copies[0] = pltpu.async_copy(src_hbm.at[pl.ds(0, CHUNK)], buf0, sem0)
        for ci in range(NCHUNK):
            if ci + 1 < NCHUNK:
                copies[(ci + 1) % 2] = pltpu.async_copy(
                    src_hbm.at[pl.ds((ci + 1) * CHUNK, CHUNK)],
                    bufs[(ci + 1) % 2],
                    sems[(ci + 1) % 2],
                )
            copies[ci % 2].wait()
            buf = bufs[ci % 2]

            def body(i, carry):
                tok = buf[pl.ds(i * 16, 16)]
                rel = tok - base
                mask = (rel >= 0) & (rel < BINS_PER_TILE)
                plsc.addupdate_scatter(hist, [rel], ones16, mask=mask)
                return carry

            lax.fori_loop(0, VREGS_PER_CHUNK, body, 0)

        pltpu.sync_copy(hist, out_hbm.at[pl.ds(base, BINS_PER_TILE)])

    return hist_kernel(src_flat)


def _tc_broadcast(hist_pad, ign):
    hist2 = hist_pad.reshape(1, VOCAB_PAD)

    def body(ign_ref, hist_ref, out_ref):
        j = pl.program_id(0)
        row = hist_ref[...]
        cols = j * BW + lax.broadcasted_iota(jnp.int32, (1, BW), 1)
        ign_v = ign_ref[0]
        row = jnp.where((cols == ign_v) | (cols == 1) | (cols == 2), 0.0, row)
        out_ref[...] = jnp.broadcast_to(row, (BATCH, BW))

    return pl.pallas_call(
        body,
        grid=(GRID,),
        in_specs=[
            pl.BlockSpec(memory_space=pltpu.SMEM),
            pl.BlockSpec((1, BW), lambda j: (0, j)),
        ],
        out_specs=pl.BlockSpec((BATCH, BW), lambda j: (0, j)),
        out_shape=jax.ShapeDtypeStruct((BATCH, VOCAB), jnp.float32),
    )(ign, hist2)


def kernel(src, ignore_index):
    src_flat = src.reshape(-1)
    hist = _sc_histogram(src_flat)
    ign = jnp.asarray(ignore_index, jnp.int32).reshape(1)
    return _tc_broadcast(hist, ign)


# speedup vs baseline: 1.9397x; 1.9397x over previous
import functools

import jax
import jax.numpy as jnp
from jax import lax
from jax.experimental import pallas as pl
from jax.experimental.pallas import tpu as pltpu
from jax.experimental.pallas import tpu_sc as plsc

VOCAB = 100000
SEQ_LEN = 200
BATCH = 1024
NTOK = SEQ_LEN * BATCH

NUM_WORKERS = 32
BINS_PER_TILE = 3200
VOCAB_PAD = NUM_WORKERS * BINS_PER_TILE

CHUNK = 8192
NCHUNK = NTOK // CHUNK
VREGS_PER_CHUNK = CHUNK // 16

BW = 2048
GRID = (VOCAB + BW - 1) // BW


def _sc_histogram(src_flat):
    mesh = plsc.VectorSubcoreMesh(core_axis_name="c", subcore_axis_name="s")

    @functools.partial(
        pl.kernel,
        mesh=mesh,
        out_type=jax.ShapeDtypeStruct((VOCAB_PAD,), jnp.float32),
        compiler_params=pltpu.CompilerParams(needs_layout_passes=False),
        scratch_types=[
            pltpu.VMEM((CHUNK,), jnp.int32),
            pltpu.VMEM((CHUNK,), jnp.int32),
            pltpu.VMEM((BINS_PER_TILE,), jnp.float32),
            pltpu.SemaphoreType.DMA,
            pltpu.SemaphoreType.DMA,
        ],
    )
    def hist_kernel(src_hbm, out_hbm, buf0, buf1, hist, sem0, sem1):
        c = lax.axis_index("c")
        s = lax.axis_index("s")
        wid = s * 2 + c
        base = wid * BINS_PER_TILE

        zeros16 = jnp.zeros((16,), jnp.float32)

        def zero_body(i, carry):
            hist[pl.ds(i * 16, 16)] = zeros16
            return carry

        lax.fori_loop(0, BINS_PER_TILE // 16, zero_body, 0)

        ones16 = jnp.ones((16,), jnp.float32)
        bufs = (buf0, buf1)
        sems = (sem0, sem1)

        copies = [None, None]
        copies[0] = pltpu.async_copy(src_hbm.at[pl.ds(0, CHUNK)], buf0, sem0)
        for ci in range(NCHUNK):
            if ci + 1 < NCHUNK:
                copies[(ci + 1) % 2] = pltpu.async_copy(
                    src_hbm.at[pl.ds((ci + 1) * CHUNK, CHUNK)],
                    bufs[(ci + 1) % 2],
                    sems[(ci + 1) % 2],
                )
            copies[ci % 2].wait()
            buf = bufs[ci % 2]

            def body(i, carry):
                tok = buf[pl.ds(i * 16, 16)]
                rel = tok - base
                mask = (rel >= 0) & (rel < BINS_PER_TILE)
                plsc.addupdate_scatter(hist, [rel], ones16, mask=mask)
                return carry

            lax.fori_loop(0, VREGS_PER_CHUNK, body, 0)

        pltpu.sync_copy(hist, out_hbm.at[pl.ds(base, BINS_PER_TILE)])

    return hist_kernel(src_flat)


def _tc_broadcast(hist_pad, ign):
    hist2 = hist_pad.reshape(1, VOCAB_PAD)

    def body(ign_ref, hist_ref, out_ref):
        j = pl.program_id(0)
        row = hist_ref[...]
        cols = j * BW + lax.broadcasted_iota(jnp.int32, (1, BW), 1)
        ign_v = ign_ref[0]
        row = jnp.where((cols == ign_v) | (cols == 1) | (cols == 2), 0.0, row)
        out_ref[...] = jnp.broadcast_to(row, (BATCH, BW))

    return pl.pallas_call(
        body,
        grid=(GRID,),
        in_specs=[
            pl.BlockSpec(memory_space=pltpu.SMEM),
            pl.BlockSpec((1, BW), lambda j: (0, j)),
        ],
        out_specs=pl.BlockSpec((BATCH, BW), lambda j: (0, j)),
        out_shape=jax.ShapeDtypeStruct((BATCH, VOCAB), jnp.float32),
    )(ign, hist2)


def kernel(src, ignore_index):
    src_flat = src.reshape(-1)
    hist = _sc_histogram(src_flat)
    ign = jnp.asarray(ignore_index, jnp.int32).reshape(1)
    return _tc_broadcast(hist, ign)


# speedup vs baseline: 2.2769x; 1.1739x over previous
import functools

import jax
import jax.numpy as jnp
from jax import lax
from jax.experimental import pallas as pl
from jax.experimental.pallas import tpu as pltpu
from jax.experimental.pallas import tpu_sc as plsc

VOCAB = 100000
SEQ_LEN = 200
BATCH = 1024
NTOK = SEQ_LEN * BATCH

TOKEN_WAYS = 8
VOCAB_WAYS = 4
VOCAB_PAD = 102400
BINS_PER_TILE = VOCAB_PAD // VOCAB_WAYS
TOK_PER_TILE = NTOK // TOKEN_WAYS

CHUNK = 6400
NCHUNK = TOK_PER_TILE // CHUNK
VREGS_PER_CHUNK = CHUNK // 16

RB = 16
NSTEP = BATCH // RB


def _sc_histogram(src_flat):
    mesh = plsc.VectorSubcoreMesh(core_axis_name="c", subcore_axis_name="s")

    @functools.partial(
        pl.kernel,
        mesh=mesh,
        out_type=jax.ShapeDtypeStruct((TOKEN_WAYS * VOCAB_PAD,), jnp.float32),
        compiler_params=pltpu.CompilerParams(needs_layout_passes=False),
        scratch_types=[
            pltpu.VMEM((CHUNK,), jnp.int32),
            pltpu.VMEM((CHUNK,), jnp.int32),
            pltpu.VMEM((BINS_PER_TILE,), jnp.float32),
            pltpu.SemaphoreType.DMA,
            pltpu.SemaphoreType.DMA,
        ],
    )
    def hist_kernel(src_hbm, out_hbm, buf0, buf1, hist, sem0, sem1):
        c = lax.axis_index("c")
        s = lax.axis_index("s")
        wid = s * 2 + c
        g = wid // VOCAB_WAYS
        v = wid % VOCAB_WAYS
        base = v * BINS_PER_TILE
        tok0 = g * TOK_PER_TILE

        zeros16 = jnp.zeros((16,), jnp.float32)

        def zero_body(i, carry):
            hist[pl.ds(i * 16, 16)] = zeros16
            return carry

        lax.fori_loop(0, BINS_PER_TILE // 16, zero_body, 0)

        ones16 = jnp.ones((16,), jnp.float32)
        bufs = (buf0, buf1)
        sems = (sem0, sem1)

        copies = [None, None]
        copies[0] = pltpu.async_copy(
            src_hbm.at[pl.ds(tok0, CHUNK)], buf0, sem0)
        for ci in range(NCHUNK):
            if ci + 1 < NCHUNK:
                copies[(ci + 1) % 2] = pltpu.async_copy(
                    src_hbm.at[pl.ds(tok0 + (ci + 1) * CHUNK, CHUNK)],
                    bufs[(ci + 1) % 2],
                    sems[(ci + 1) % 2],
                )
            copies[ci % 2].wait()
            buf = bufs[ci % 2]

            def body(i, carry):
                tok = buf[pl.ds(i * 16, 16)]
                rel = tok - base
                mask = (rel >= 0) & (rel < BINS_PER_TILE)
                plsc.addupdate_scatter(hist, [rel], ones16, mask=mask)
                return carry

            lax.fori_loop(0, VREGS_PER_CHUNK, body, 0)

        pltpu.sync_copy(
            hist, out_hbm.at[pl.ds(g * VOCAB_PAD + base, BINS_PER_TILE)])

    return hist_kernel(src_flat).reshape(TOKEN_WAYS, VOCAB_PAD)


def _tc_broadcast(hist_parts, ign):
    def body(ign_ref, hist_ref, out_ref, row):
        @pl.when(pl.program_id(0) == 0)
        def _():
            summed = jnp.sum(hist_ref[...], axis=0, keepdims=True)
            cols = lax.broadcasted_iota(jnp.int32, (1, VOCAB_PAD), 1)
            ign_v = ign_ref[0]
            keep = (cols == ign_v) | (cols == 1) | (cols == 2)
            row[...] = jnp.where(keep, 0.0, summed)

        out_ref[...] = jnp.broadcast_to(row[:, :VOCAB], (RB, VOCAB))

    return pl.pallas_call(
        body,
        grid=(NSTEP,),
        in_specs=[
            pl.BlockSpec(memory_space=pltpu.SMEM),
            pl.BlockSpec((TOKEN_WAYS, VOCAB_PAD), lambda j: (0, 0)),
        ],
        out_specs=pl.BlockSpec((RB, VOCAB), lambda j: (j, 0)),
        out_shape=jax.ShapeDtypeStruct((BATCH, VOCAB), jnp.float32),
        scratch_shapes=[pltpu.VMEM((1, VOCAB_PAD), jnp.float32)],
    )(ign, hist_parts)


def kernel(src, ignore_index):
    src_flat = src.reshape(-1)
    hist_parts = _sc_histogram(src_flat)
    ign = jnp.asarray(ignore_index, jnp.int32).reshape(1)
    return _tc_broadcast(hist_parts, ign)
